# baseline (speedup 1.0000x reference)
import functools

import jax
import jax.numpy as jnp
from jax import lax
from jax.experimental import pallas as pl
from jax.experimental.pallas import tpu as pltpu
from jax.experimental.pallas import tpu_sc as plsc


TM = 128
NC = 2
NS = 16
NW = NC * NS
LANES = 16


def _wid():
    return lax.axis_index("s") * NC + lax.axis_index("c")


def _dispatch_body(idx_hbm, hid_hbm, out_hbm, idx_v, buf, sem, *, rows_w):
    base = _wid() * rows_w
    pltpu.sync_copy(idx_hbm.at[pl.ds(base, rows_w)], idx_v)
    nch = (rows_w + 127) // 128
    ch = rows_w // nch
    cps = [
        pltpu.async_copy(hid_hbm.at[idx_v.at[pl.ds(c * ch, ch)]],
                         buf.at[pl.ds(c * ch, ch)], sem)
        for c in range(nch)
    ]
    for cp in cps:
        cp.wait()
    pltpu.sync_copy(buf, out_hbm.at[pl.ds(base, rows_w)])


def _sc_dispatch(row_token, hidden):
    p_pad = row_token.shape[0]
    k_dim = hidden.shape[1]
    rows_w = p_pad // NW
    mesh = plsc.VectorSubcoreMesh(core_axis_name="c", subcore_axis_name="s")
    f = pl.kernel(
        functools.partial(_dispatch_body, rows_w=rows_w),
        out_type=jax.ShapeDtypeStruct((p_pad, k_dim), jnp.float32),
        mesh=mesh,
        scratch_types=[
            pltpu.VMEM((rows_w,), jnp.int32),
            pltpu.VMEM((rows_w, k_dim), jnp.float32),
            pltpu.SemaphoreType.DMA,
        ],
    )
    return f(row_token, hidden)


def _finalize_body(ridx_hbm, y_hbm, out_hbm, idx0, idx1, b0, b1, s0, s1,
                   *, tok_w, k_dim, topk):
    base = _wid() * tok_w
    pltpu.sync_copy(ridx_hbm.at[0, pl.ds(base, tok_w)], idx0)
    cp0 = pltpu.async_copy(y_hbm.at[idx0], b0, s0)
    for s in range(1, topk):
        pltpu.sync_copy(ridx_hbm.at[s, pl.ds(base, tok_w)], idx1)
        cp1 = pltpu.async_copy(y_hbm.at[idx1], b1, s1)
        if s == 1:
            cp0.wait()
        cp1.wait()

        def row_body(i, carry):
            for j in range(k_dim // LANES):
                sl = pl.ds(j * LANES, LANES)
                plsc.addupdate(b0.at[i, sl], b1[i, sl])
            return carry

        lax.fori_loop(0, tok_w, row_body, 0)
    pltpu.sync_copy(b0, out_hbm.at[pl.ds(base, tok_w)])


def _sc_finalize(ridx, y_rows, m):
    topk = ridx.shape[0]
    k_dim = y_rows.shape[1]
    tok_w = m // NW
    mesh = plsc.VectorSubcoreMesh(core_axis_name="c", subcore_axis_name="s")
    f = pl.kernel(
        functools.partial(_finalize_body, tok_w=tok_w, k_dim=k_dim, topk=topk),
        out_type=jax.ShapeDtypeStruct((m, k_dim), jnp.float32),
        mesh=mesh,
        scratch_types=[
            pltpu.VMEM((tok_w,), jnp.int32),
            pltpu.VMEM((tok_w,), jnp.int32),
            pltpu.VMEM((tok_w, k_dim), jnp.float32),
            pltpu.VMEM((tok_w, k_dim), jnp.float32),
            pltpu.SemaphoreType.DMA,
            pltpu.SemaphoreType.DMA,
        ],
    )
    return f(ridx, y_rows)


def _gemm_tile(te_ref, x_ref, w1_ref, w2_ref, wt_ref, y_ref, *, n_inter):
    x = x_ref[...]
    w1 = w1_ref[0]
    h = jnp.dot(x, w1, preferred_element_type=jnp.float32)
    gate = h[:, :n_inter]
    up = h[:, n_inter:]
    act = (gate * jax.nn.sigmoid(gate)) * up
    y = jnp.dot(act, w2_ref[0],
                preferred_element_type=jnp.float32)
    y_ref[...] = y * wt_ref[...]


def _grouped_gemm(x_rows, w1, w2, row_weight, tile_expert):
    p_pad, k_dim = x_rows.shape
    e_dim, _, n2 = w1.shape
    n_inter = w2.shape[1]
    tiles = p_pad // TM
    grid_spec = pltpu.PrefetchScalarGridSpec(
        num_scalar_prefetch=1,
        grid=(tiles,),
        in_specs=[
            pl.BlockSpec((TM, k_dim), lambda t, te: (t, 0)),
            pl.BlockSpec((1, k_dim, n2), lambda t, te: (te[t], 0, 0)),
            pl.BlockSpec((1, n_inter, k_dim), lambda t, te: (te[t], 0, 0)),
            pl.BlockSpec((TM, 1), lambda t, te: (t, 0)),
        ],
        out_specs=pl.BlockSpec((TM, k_dim), lambda t, te: (t, 0)),
    )
    return pl.pallas_call(
        functools.partial(_gemm_tile, n_inter=n_inter),
        grid_spec=grid_spec,
        out_shape=jax.ShapeDtypeStruct((p_pad, k_dim), jnp.float32),
    )(tile_expert, x_rows, w1, w2, row_weight)


def kernel(hidden_states, w1, w2, topk_weights, topk_ids):
    m, k_dim = hidden_states.shape
    e_dim = w1.shape[0]
    topk = topk_ids.shape[1]
    p = m * topk
    p_pad = p + e_dim * TM

    e_flat = topk_ids.reshape(p)
    onehot = (e_flat[None, :] == jnp.arange(e_dim, dtype=jnp.int32)[:, None])
    onehot = onehot.astype(jnp.int32)
    csum = jnp.cumsum(onehot, axis=1)
    counts = csum[:, -1]
    rank = jnp.sum(onehot * csum, axis=0) - 1
    pg = ((counts + TM - 1) // TM) * TM
    pg_cum = jnp.cumsum(pg)
    pstart = pg_cum - pg
    dest = (pstart[e_flat] + rank).astype(jnp.int32)
    row_token = (jnp.arange(p_pad, dtype=jnp.int32) % m).at[dest].set(
        (jnp.arange(p, dtype=jnp.int32) // topk))
    row_weight = jnp.zeros((p_pad, 1), jnp.float32).at[dest, 0].set(
        topk_weights.reshape(p))
    row_of_pair = dest
    tiles = p_pad // TM
    tile_expert = jnp.sum(
        (jnp.arange(tiles, dtype=jnp.int32)[:, None] * TM >= pg_cum[None, :])
        .astype(jnp.int32), axis=1)
    tile_expert = jnp.minimum(tile_expert, e_dim - 1).astype(jnp.int32)

    x_rows = _sc_dispatch(row_token, hidden_states)

    y_rows = _grouped_gemm(x_rows, w1, w2, row_weight, tile_expert)

    ridx = row_of_pair.reshape(m, topk).T
    out = _sc_finalize(ridx, y_rows, m)
    return out


# speedup vs baseline: 1.1036x; 1.1036x over previous
import functools

import jax
import jax.numpy as jnp
from jax import lax
from jax.experimental import pallas as pl
from jax.experimental.pallas import tpu as pltpu
from jax.experimental.pallas import tpu_sc as plsc


TM = 128
NC = 2
NS = 16
NW = NC * NS
LANES = 16


def _wid():
    return lax.axis_index("s") * NC + lax.axis_index("c")


def _dispatch_body(ridx_hbm, hid_hbm, out_hbm, idx0, idx1, buf, sem,
                   *, tok_w, topk):
    base = _wid() * tok_w
    pltpu.sync_copy(hid_hbm.at[pl.ds(base, tok_w)], buf)
    idxs = (idx0, idx1)
    cps = []
    for s in range(topk):
        pltpu.sync_copy(ridx_hbm.at[s, pl.ds(base, tok_w)], idxs[s % 2])
        cps.append(pltpu.async_copy(buf, out_hbm.at[idxs[s % 2]], sem))
        if s % 2 == 1:
            cps[s - 1].wait()
            cps[s].wait()
    for s in range(2 * (topk // 2), topk):
        cps[s].wait()


def _sc_dispatch(ridx, hidden, p_pad):
    m, k_dim = hidden.shape
    topk = ridx.shape[0]
    tok_w = m // NW
    mesh = plsc.VectorSubcoreMesh(core_axis_name="c", subcore_axis_name="s")
    f = pl.kernel(
        functools.partial(_dispatch_body, tok_w=tok_w, topk=topk),
        out_type=jax.ShapeDtypeStruct((p_pad, k_dim), jnp.float32),
        mesh=mesh,
        scratch_types=[
            pltpu.VMEM((tok_w,), jnp.int32),
            pltpu.VMEM((tok_w,), jnp.int32),
            pltpu.VMEM((tok_w, k_dim), jnp.float32),
            pltpu.SemaphoreType.DMA,
        ],
    )
    return f(ridx, hidden)


def _finalize_body(ridx_hbm, y_hbm, out_hbm, idx0, idx1, b0, b1, s0, s1,
                   *, tok_w, k_dim, topk):
    base = _wid() * tok_w
    pltpu.sync_copy(ridx_hbm.at[0, pl.ds(base, tok_w)], idx0)
    cp0 = pltpu.async_copy(y_hbm.at[idx0], b0, s0)
    for s in range(1, topk):
        pltpu.sync_copy(ridx_hbm.at[s, pl.ds(base, tok_w)], idx1)
        cp1 = pltpu.async_copy(y_hbm.at[idx1], b1, s1)
        if s == 1:
            cp0.wait()
        cp1.wait()

        def row_body(i, carry):
            for j in range(k_dim // LANES):
                sl = pl.ds(j * LANES, LANES)
                plsc.addupdate(b0.at[i, sl], b1[i, sl])
            return carry

        lax.fori_loop(0, tok_w, row_body, 0)
    pltpu.sync_copy(b0, out_hbm.at[pl.ds(base, tok_w)])


def _sc_finalize(ridx, y_rows, m):
    topk = ridx.shape[0]
    k_dim = y_rows.shape[1]
    tok_w = m // NW
    mesh = plsc.VectorSubcoreMesh(core_axis_name="c", subcore_axis_name="s")
    f = pl.kernel(
        functools.partial(_finalize_body, tok_w=tok_w, k_dim=k_dim, topk=topk),
        out_type=jax.ShapeDtypeStruct((m, k_dim), jnp.float32),
        mesh=mesh,
        scratch_types=[
            pltpu.VMEM((tok_w,), jnp.int32),
            pltpu.VMEM((tok_w,), jnp.int32),
            pltpu.VMEM((tok_w, k_dim), jnp.float32),
            pltpu.VMEM((tok_w, k_dim), jnp.float32),
            pltpu.SemaphoreType.DMA,
            pltpu.SemaphoreType.DMA,
        ],
    )
    return f(ridx, y_rows)


def _gemm_tile(te_ref, x_ref, w1_ref, w2_ref, wt_ref, y_ref, *, n_inter):
    x = x_ref[...]
    w1 = w1_ref[0]
    h = jnp.dot(x, w1, preferred_element_type=jnp.float32)
    gate = h[:, :n_inter]
    up = h[:, n_inter:]
    act = (gate * jax.nn.sigmoid(gate)) * up
    y = jnp.dot(act, w2_ref[0],
                preferred_element_type=jnp.float32)
    y_ref[...] = y * wt_ref[...]


def _grouped_gemm(x_rows, w1, w2, row_weight, tile_expert):
    p_pad, k_dim = x_rows.shape
    e_dim, _, n2 = w1.shape
    n_inter = w2.shape[1]
    tiles = p_pad // TM
    grid_spec = pltpu.PrefetchScalarGridSpec(
        num_scalar_prefetch=1,
        grid=(tiles,),
        in_specs=[
            pl.BlockSpec((TM, k_dim), lambda t, te: (t, 0)),
            pl.BlockSpec((1, k_dim, n2), lambda t, te: (te[t], 0, 0)),
            pl.BlockSpec((1, n_inter, k_dim), lambda t, te: (te[t], 0, 0)),
            pl.BlockSpec((TM, 1), lambda t, te: (t, 0)),
        ],
        out_specs=pl.BlockSpec((TM, k_dim), lambda t, te: (t, 0)),
    )
    return pl.pallas_call(
        functools.partial(_gemm_tile, n_inter=n_inter),
        grid_spec=grid_spec,
        out_shape=jax.ShapeDtypeStruct((p_pad, k_dim), jnp.float32),
    )(tile_expert, x_rows, w1, w2, row_weight)


def kernel(hidden_states, w1, w2, topk_weights, topk_ids):
    m, k_dim = hidden_states.shape
    e_dim = w1.shape[0]
    topk = topk_ids.shape[1]
    p = m * topk
    p_pad = p + e_dim * TM

    e_flat = topk_ids.reshape(p)
    onehot = (e_flat[None, :] == jnp.arange(e_dim, dtype=jnp.int32)[:, None])
    onehot = onehot.astype(jnp.int32)
    csum = jnp.cumsum(onehot, axis=1)
    counts = csum[:, -1]
    rank = jnp.sum(onehot * csum, axis=0) - 1
    pg = ((counts + TM - 1) // TM) * TM
    pg_cum = jnp.cumsum(pg)
    pstart = pg_cum - pg
    dest = (pstart[e_flat] + rank).astype(jnp.int32)
    row_weight = jnp.zeros((p_pad, 1), jnp.float32).at[dest, 0].set(
        topk_weights.reshape(p))
    tiles = p_pad // TM
    tile_expert = jnp.sum(
        (jnp.arange(tiles, dtype=jnp.int32)[:, None] * TM >= pg_cum[None, :])
        .astype(jnp.int32), axis=1)
    tile_expert = jnp.minimum(tile_expert, e_dim - 1).astype(jnp.int32)

    ridx = dest.reshape(m, topk).T
    x_rows = _sc_dispatch(ridx, hidden_states, p_pad)

    y_rows = _grouped_gemm(x_rows, w1, w2, row_weight, tile_expert)

    out = _sc_finalize(ridx, y_rows, m)
    return out


# speedup vs baseline: 1.1821x; 1.0712x over previous
import functools

import jax
import jax.numpy as jnp
from jax import lax
from jax.experimental import pallas as pl
from jax.experimental.pallas import tpu as pltpu
from jax.experimental.pallas import tpu_sc as plsc


TM = 128
NC = 2
NS = 16
NW = NC * NS
LANES = 16


def _wid():
    return lax.axis_index("s") * NC + lax.axis_index("c")


def _dispatch_body(ridx_hbm, hid_hbm, out_hbm, idx0, idx1, buf, sem,
                   *, tok_w, topk):
    base = _wid() * tok_w
    pltpu.sync_copy(hid_hbm.at[pl.ds(base, tok_w)], buf)
    idxs = (idx0, idx1)
    cps = []
    for s in range(topk):
        pltpu.sync_copy(ridx_hbm.at[s, pl.ds(base, tok_w)], idxs[s % 2])
        cps.append(pltpu.async_copy(buf, out_hbm.at[idxs[s % 2]], sem))
        if s % 2 == 1:
            cps[s - 1].wait()
            cps[s].wait()
    for s in range(2 * (topk // 2), topk):
        cps[s].wait()


def _sc_dispatch(ridx, hidden, p_pad):
    m, k_dim = hidden.shape
    topk = ridx.shape[0]
    tok_w = m // NW
    mesh = plsc.VectorSubcoreMesh(core_axis_name="c", subcore_axis_name="s")
    f = pl.kernel(
        functools.partial(_dispatch_body, tok_w=tok_w, topk=topk),
        out_type=jax.ShapeDtypeStruct((p_pad, k_dim), jnp.float32),
        mesh=mesh,
        scratch_types=[
            pltpu.VMEM((tok_w,), jnp.int32),
            pltpu.VMEM((tok_w,), jnp.int32),
            pltpu.VMEM((tok_w, k_dim), jnp.float32),
            pltpu.SemaphoreType.DMA,
        ],
    )
    return f(ridx, hidden)


def _finalize_body(ridx_hbm, wtt_hbm, y_hbm, out_hbm,
                   idx0, idx1, wt0, wt1, b0, b1, s0, s1,
                   *, tok_w, k_dim, topk):
    base = _wid() * tok_w
    pltpu.sync_copy(ridx_hbm.at[0, pl.ds(base, tok_w)], idx0)
    pltpu.sync_copy(wtt_hbm.at[0, pl.ds(base, tok_w)], wt0)
    cp0 = pltpu.async_copy(y_hbm.at[idx0], b0, s0)
    for s in range(1, topk):
        pltpu.sync_copy(ridx_hbm.at[s, pl.ds(base, tok_w)], idx1)
        pltpu.sync_copy(wtt_hbm.at[s, pl.ds(base, tok_w)], wt1)
        cp1 = pltpu.async_copy(y_hbm.at[idx1], b1, s1)
        if s == 1:
            cp0.wait()
        cp1.wait()
        first = s == 1

        def row_body(i, carry):
            w1b = wt1[i, :]
            if first:
                w0b = wt0[i, :]
            for j in range(k_dim // LANES):
                sl = pl.ds(j * LANES, LANES)
                if first:
                    b0[i, sl] = b0[i, sl] * w0b + b1[i, sl] * w1b
                else:
                    b0[i, sl] = b0[i, sl] + b1[i, sl] * w1b
            return carry

        lax.fori_loop(0, tok_w, row_body, 0)
    pltpu.sync_copy(b0, out_hbm.at[pl.ds(base, tok_w)])


def _sc_finalize(ridx, wtt, y_rows, m):
    topk = ridx.shape[0]
    k_dim = y_rows.shape[1]
    tok_w = m // NW
    mesh = plsc.VectorSubcoreMesh(core_axis_name="c", subcore_axis_name="s")
    f = pl.kernel(
        functools.partial(_finalize_body, tok_w=tok_w, k_dim=k_dim, topk=topk),
        out_type=jax.ShapeDtypeStruct((m, k_dim), jnp.float32),
        mesh=mesh,
        scratch_types=[
            pltpu.VMEM((tok_w,), jnp.int32),
            pltpu.VMEM((tok_w,), jnp.int32),
            pltpu.VMEM((tok_w, LANES), jnp.float32),
            pltpu.VMEM((tok_w, LANES), jnp.float32),
            pltpu.VMEM((tok_w, k_dim), jnp.float32),
            pltpu.VMEM((tok_w, k_dim), jnp.float32),
            pltpu.SemaphoreType.DMA,
            pltpu.SemaphoreType.DMA,
        ],
    )
    return f(ridx, wtt, y_rows)


def _gemm_tile(te_ref, x_ref, w1_ref, w2_ref, y_ref, *, n_inter):
    x = x_ref[...]
    w1 = w1_ref[0]
    h = jnp.dot(x, w1, preferred_element_type=jnp.float32)
    gate = h[:, :n_inter]
    up = h[:, n_inter:]
    act = (gate * jax.nn.sigmoid(gate)) * up
    y_ref[...] = jnp.dot(act, w2_ref[0],
                         preferred_element_type=jnp.float32)


def _grouped_gemm(x_rows, w1, w2, tile_expert):
    p_pad, k_dim = x_rows.shape
    e_dim, _, n2 = w1.shape
    n_inter = w2.shape[1]
    tiles = p_pad // TM
    grid_spec = pltpu.PrefetchScalarGridSpec(
        num_scalar_prefetch=1,
        grid=(tiles,),
        in_specs=[
            pl.BlockSpec((TM, k_dim), lambda t, te: (t, 0)),
            pl.BlockSpec((1, k_dim, n2), lambda t, te: (te[t], 0, 0)),
            pl.BlockSpec((1, n_inter, k_dim), lambda t, te: (te[t], 0, 0)),
        ],
        out_specs=pl.BlockSpec((TM, k_dim), lambda t, te: (t, 0)),
    )
    return pl.pallas_call(
        functools.partial(_gemm_tile, n_inter=n_inter),
        grid_spec=grid_spec,
        out_shape=jax.ShapeDtypeStruct((p_pad, k_dim), jnp.float32),
    )(tile_expert, x_rows, w1, w2)


def kernel(hidden_states, w1, w2, topk_weights, topk_ids):
    m, k_dim = hidden_states.shape
    e_dim = w1.shape[0]
    topk = topk_ids.shape[1]
    p = m * topk
    p_pad = p + e_dim * TM

    e_flat = topk_ids.reshape(p)
    oh = (e_flat[None, :] == jnp.arange(e_dim, dtype=jnp.int32)[:, None])
    oh = oh.astype(jnp.float32)
    cb = 128
    nch = p // cb
    ohc = oh.reshape(e_dim, nch, cb)
    tri_in = jnp.triu(jnp.ones((cb, cb), jnp.float32))
    within = jnp.matmul(ohc, tri_in)
    totals = within[:, :, -1]
    tri_ex = jnp.triu(jnp.ones((nch, nch), jnp.float32), k=1)
    offs = jnp.matmul(totals, tri_ex)
    rank_in = (within + offs[:, :, None]).reshape(e_dim, p)
    rank = jnp.sum(oh * rank_in, axis=0) - 1.0
    counts = jnp.sum(totals, axis=1).astype(jnp.int32)
    pg = ((counts + TM - 1) // TM) * TM
    pg_cum = jnp.cumsum(pg)
    pstart = pg_cum - pg
    dest = (pstart[e_flat].astype(jnp.float32) + rank).astype(jnp.int32)
    tiles = p_pad // TM
    tile_expert = jnp.sum(
        (jnp.arange(tiles, dtype=jnp.int32)[:, None] * TM >= pg_cum[None, :])
        .astype(jnp.int32), axis=1)
    tile_expert = jnp.minimum(tile_expert, e_dim - 1).astype(jnp.int32)

    ridx = dest.reshape(m, topk).T
    x_rows = _sc_dispatch(ridx, hidden_states, p_pad)

    y_rows = _grouped_gemm(x_rows, w1, w2, tile_expert)

    wtt = jnp.broadcast_to(topk_weights.T[:, :, None], (topk, m, LANES))
    out = _sc_finalize(ridx, wtt, y_rows, m)
    return out


# speedup vs baseline: 1.2199x; 1.0320x over previous
import functools

import jax
import jax.numpy as jnp
from jax import lax
from jax.experimental import pallas as pl
from jax.experimental.pallas import tpu as pltpu
from jax.experimental.pallas import tpu_sc as plsc


TM = 128
NC = 2
NS = 16
NW = NC * NS
LANES = 16


def _wid():
    return lax.axis_index("s") * NC + lax.axis_index("c")


def _dispatch_body(ridx_hbm, hid_hbm, out_hbm, idx0, idx1, buf, sem,
                   *, tok_w, topk):
    base = _wid() * tok_w
    pltpu.sync_copy(hid_hbm.at[pl.ds(base, tok_w)], buf)
    idxs = (idx0, idx1)
    cps = []
    for s in range(topk):
        pltpu.sync_copy(ridx_hbm.at[s, pl.ds(base, tok_w)], idxs[s % 2])
        cps.append(pltpu.async_copy(buf, out_hbm.at[idxs[s % 2]], sem))
        if s % 2 == 1:
            cps[s - 1].wait()
            cps[s].wait()
    for s in range(2 * (topk // 2), topk):
        cps[s].wait()


def _sc_dispatch(ridx, hidden, p_pad):
    m, k_dim = hidden.shape
    topk = ridx.shape[0]
    tok_w = m // NW
    mesh = plsc.VectorSubcoreMesh(core_axis_name="c", subcore_axis_name="s")
    f = pl.kernel(
        functools.partial(_dispatch_body, tok_w=tok_w, topk=topk),
        out_type=jax.ShapeDtypeStruct((p_pad, k_dim), jnp.float32),
        mesh=mesh,
        scratch_types=[
            pltpu.VMEM((tok_w,), jnp.int32),
            pltpu.VMEM((tok_w,), jnp.int32),
            pltpu.VMEM((tok_w, k_dim), jnp.float32),
            pltpu.SemaphoreType.DMA,
        ],
    )
    return f(ridx, hidden)


def _finalize_body(ridx_hbm, wtt_hbm, y_hbm, out_hbm,
                   idx0, idx1, wt0, wt1, b0, b1, s0, s1,
                   *, tok_w, k_dim, topk):
    base = _wid() * tok_w
    pltpu.sync_copy(ridx_hbm.at[0, pl.ds(base, tok_w)], idx0)
    pltpu.sync_copy(wtt_hbm.at[0, pl.ds(base, tok_w)], wt0)
    cp0 = pltpu.async_copy(y_hbm.at[idx0], b0, s0)
    for s in range(1, topk):
        pltpu.sync_copy(ridx_hbm.at[s, pl.ds(base, tok_w)], idx1)
        pltpu.sync_copy(wtt_hbm.at[s, pl.ds(base, tok_w)], wt1)
        cp1 = pltpu.async_copy(y_hbm.at[idx1], b1, s1)
        if s == 1:
            cp0.wait()
        cp1.wait()
        first = s == 1

        def row_body(i, carry):
            w1b = wt1[i, :]
            if first:
                w0b = wt0[i, :]
            for j in range(k_dim // LANES):
                sl = pl.ds(j * LANES, LANES)
                if first:
                    b0[i, sl] = b0[i, sl] * w0b + b1[i, sl] * w1b
                else:
                    b0[i, sl] = b0[i, sl] + b1[i, sl] * w1b
            return carry

        lax.fori_loop(0, tok_w, row_body, 0)
    pltpu.sync_copy(b0, out_hbm.at[pl.ds(base, tok_w)])


def _sc_finalize(ridx, wtt, y_rows, m):
    topk = ridx.shape[0]
    k_dim = y_rows.shape[1]
    tok_w = m // NW
    mesh = plsc.VectorSubcoreMesh(core_axis_name="c", subcore_axis_name="s")
    f = pl.kernel(
        functools.partial(_finalize_body, tok_w=tok_w, k_dim=k_dim, topk=topk),
        out_type=jax.ShapeDtypeStruct((m, k_dim), jnp.float32),
        mesh=mesh,
        scratch_types=[
            pltpu.VMEM((tok_w,), jnp.int32),
            pltpu.VMEM((tok_w,), jnp.int32),
            pltpu.VMEM((tok_w, LANES), jnp.float32),
            pltpu.VMEM((tok_w, LANES), jnp.float32),
            pltpu.VMEM((tok_w, k_dim), jnp.float32),
            pltpu.VMEM((tok_w, k_dim), jnp.float32),
            pltpu.SemaphoreType.DMA,
            pltpu.SemaphoreType.DMA,
        ],
    )
    return f(ridx, wtt, y_rows)


def _meta_body(er_ref, dest_ref, te_ref, *, e_dim, nch, cb, tm):
    rows = e_dim * nch
    er = er_ref[...]
    e_of_row = lax.broadcasted_iota(jnp.int32, (rows, cb), 0) // nch
    er_t = jnp.concatenate([er] * e_dim, axis=0)
    oh = (er_t == e_of_row).astype(jnp.float32)
    ri = lax.broadcasted_iota(jnp.int32, (cb, cb), 0)
    ci = lax.broadcasted_iota(jnp.int32, (cb, cb), 1)
    tri_in = (ri <= ci).astype(jnp.float32)
    within = jnp.dot(oh, tri_in, preferred_element_type=jnp.float32)
    totals = within[:, cb - 1:cb]
    rr = lax.broadcasted_iota(jnp.int32, (rows, rows), 0)
    cc = lax.broadcasted_iota(jnp.int32, (rows, rows), 1)
    same_e = (rr // nch) == (cc // nch)
    tri_blk = (same_e & ((cc % nch) < (rr % nch))).astype(jnp.float32)
    offs = jnp.dot(tri_blk, totals, preferred_element_type=jnp.float32)
    sum_blk = same_e.astype(jnp.float32)
    counts = jnp.dot(sum_blk, totals, preferred_element_type=jnp.float32)
    pg = jnp.floor((counts + (tm - 1)) / tm) * tm
    ex_blk = (((cc % nch) == 0) &
              ((cc // nch) < (rr // nch))).astype(jnp.float32)
    pstart = jnp.dot(ex_blk, pg, preferred_element_type=jnp.float32)
    dest_full = (within + offs + pstart - 1.0) * oh
    cr = lax.broadcasted_iota(jnp.int32, (nch, rows), 0)
    rc = lax.broadcasted_iota(jnp.int32, (nch, rows), 1)
    collapse = ((rc % nch) == cr).astype(jnp.float32)
    dest = jnp.dot(collapse, dest_full, preferred_element_type=jnp.float32,
                   precision=lax.Precision.HIGHEST)
    dest_ref[...] = dest.astype(jnp.int32)
    pg_cum = pstart + pg
    lanes = (lax.broadcasted_iota(jnp.int32, (rows, cb), 1) * tm
             ).astype(jnp.float32)
    is_rep = (lax.broadcasted_iota(jnp.int32, (rows, cb), 0) % nch) == 0
    ind = jnp.where(is_rep & (lanes >= pg_cum), 1.0, 0.0)
    te = jnp.minimum(jnp.sum(ind, axis=0), float(e_dim - 1))
    te_ref[...] = te[None, :].astype(jnp.int32)


def _routing_meta(e_r, e_dim):
    nch, cb = e_r.shape
    return pl.pallas_call(
        functools.partial(_meta_body, e_dim=e_dim, nch=nch, cb=cb, tm=TM),
        out_shape=(jax.ShapeDtypeStruct((nch, cb), jnp.int32),
                   jax.ShapeDtypeStruct((1, cb), jnp.int32)),
    )(e_r)


def _gemm_tile(te_ref, x_ref, w1_ref, w2_ref, y_ref, *, n_inter):
    x = x_ref[...]
    w1 = w1_ref[0]
    h = jnp.dot(x, w1, preferred_element_type=jnp.float32)
    gate = h[:, :n_inter]
    up = h[:, n_inter:]
    act = (gate * jax.nn.sigmoid(gate)) * up
    y_ref[...] = jnp.dot(act, w2_ref[0],
                         preferred_element_type=jnp.float32)


def _grouped_gemm(x_rows, w1, w2, tile_expert):
    p_pad, k_dim = x_rows.shape
    e_dim, _, n2 = w1.shape
    n_inter = w2.shape[1]
    tiles = p_pad // TM
    grid_spec = pltpu.PrefetchScalarGridSpec(
        num_scalar_prefetch=1,
        grid=(tiles,),
        in_specs=[
            pl.BlockSpec((TM, k_dim), lambda t, te: (t, 0)),
            pl.BlockSpec((1, k_dim, n2), lambda t, te: (te[t], 0, 0)),
            pl.BlockSpec((1, n_inter, k_dim), lambda t, te: (te[t], 0, 0)),
        ],
        out_specs=pl.BlockSpec((TM, k_dim), lambda t, te: (t, 0)),
    )
    return pl.pallas_call(
        functools.partial(_gemm_tile, n_inter=n_inter),
        grid_spec=grid_spec,
        out_shape=jax.ShapeDtypeStruct((p_pad, k_dim), jnp.float32),
    )(tile_expert, x_rows, w1, w2)


def kernel(hidden_states, w1, w2, topk_weights, topk_ids):
    m, k_dim = hidden_states.shape
    e_dim = w1.shape[0]
    topk = topk_ids.shape[1]
    p = m * topk
    p_pad = p + e_dim * TM

    tiles = p_pad // TM
    e_r = topk_ids.reshape(p // 128, 128)
    dest2d, te2d = _routing_meta(e_r, e_dim)
    dest = dest2d.reshape(p)
    tile_expert = te2d.reshape(128)[:tiles]

    ridx = dest.reshape(m, topk).T
    x_rows = _sc_dispatch(ridx, hidden_states, p_pad)

    y_rows = _grouped_gemm(x_rows, w1, w2, tile_expert)

    wtt = jnp.broadcast_to(topk_weights.T[:, :, None], (topk, m, LANES))
    out = _sc_finalize(ridx, wtt, y_rows, m)
    return out
